# baseline (speedup 1.0000x reference)
import jax
import jax.numpy as jnp
from jax import lax
from jax.experimental import pallas as pl
from jax.experimental.pallas import tpu as pltpu
from jax.experimental.pallas import tpu_sc as plsc

CTX = 50
BATCH = 4096
NINP = 64
LANES = 16
NUM_WORKERS = 32
BPW = BATCH // NUM_WORKERS
NBLK = BPW // LANES
PAIRED = 2 * NINP


def _cbow_body(phys_hbm, off_hbm, table_hbm, out_hbm, phys_v, off_v, buf0, buf1,
               acc, sem0, sem1):
    wid = lax.axis_index("s") * 2 + lax.axis_index("c")
    base = wid * BPW

    pltpu.sync_copy(phys_hbm.at[:, pl.ds(base, BPW)], phys_v)
    pltpu.sync_copy(off_hbm.at[:, pl.ds(base, BPW)], off_v)

    zeros = jnp.zeros((LANES,), jnp.float32)

    def zero_body(j, _):
        for k in range(NBLK):
            acc[j, pl.ds(k * LANES, LANES)] = zeros
        return 0

    lax.fori_loop(0, NINP, zero_body, 0)

    pltpu.async_copy(table_hbm.at[phys_v.at[0]], buf0, sem0)
    pltpu.async_copy(table_hbm.at[phys_v.at[1]], buf1, sem1)

    lanes_iota = lax.iota(jnp.int32, LANES)

    def accumulate(buf, c):
        def blk_body(blk, _):
            r0 = blk * LANES
            rows = r0 + lanes_iota
            cols0 = off_v[c, pl.ds(r0, LANES)]
            for j in range(NINP):
                val = plsc.load_gather(buf, [rows, cols0 + j])
                plsc.addupdate(acc.at[j, pl.ds(r0, LANES)], val)
            return 0

        lax.fori_loop(0, NBLK, blk_body, 0)

    def step(i, _):
        pltpu.make_async_copy(table_hbm.at[phys_v.at[0]], buf0, sem0).wait()
        accumulate(buf0, 2 * i)

        @pl.when(i < (CTX // 2) - 1)
        def _():
            pltpu.async_copy(table_hbm.at[phys_v.at[2 * i + 2]], buf0, sem0)

        pltpu.make_async_copy(table_hbm.at[phys_v.at[1]], buf1, sem1).wait()
        accumulate(buf1, 2 * i + 1)

        @pl.when(i < (CTX // 2) - 1)
        def _():
            pltpu.async_copy(table_hbm.at[phys_v.at[2 * i + 3]], buf1, sem1)

        return 0

    lax.fori_loop(0, CTX // 2, step, 0)

    pltpu.sync_copy(acc, out_hbm.at[:, pl.ds(base, BPW)])


@jax.jit
def _cbow(inp, table):
    phys = lax.shift_right_logical(inp, 1)
    off = lax.shift_left(jnp.bitwise_and(inp, 1), 6)
    table2 = table.reshape(table.shape[0] // 2, PAIRED)
    mesh = plsc.VectorSubcoreMesh(core_axis_name="c", subcore_axis_name="s")
    out_t = pl.kernel(
        _cbow_body,
        out_type=jax.ShapeDtypeStruct((NINP, BATCH), jnp.float32),
        mesh=mesh,
        scratch_types=[
            pltpu.VMEM((CTX, BPW), jnp.int32),
            pltpu.VMEM((CTX, BPW), jnp.int32),
            pltpu.VMEM((BPW, PAIRED), jnp.float32),
            pltpu.VMEM((BPW, PAIRED), jnp.float32),
            pltpu.VMEM((NINP, BPW), jnp.float32),
            pltpu.SemaphoreType.DMA,
            pltpu.SemaphoreType.DMA,
        ],
        compiler_params=pltpu.CompilerParams(needs_layout_passes=False),
    )(phys, off, table2)
    return out_t.T


def kernel(input, hidden, table):
    del hidden
    return _cbow(input.astype(jnp.int32), table)


# speedup vs baseline: 1.4134x; 1.4134x over previous
import jax
import jax.numpy as jnp
from jax import lax
from jax.experimental import pallas as pl
from jax.experimental.pallas import tpu as pltpu
from jax.experimental.pallas import tpu_sc as plsc

CTX = 50
BATCH = 4096
NINP = 64
LANES = 16
NUM_WORKERS = 32
BPW = BATCH // NUM_WORKERS
CHUNKS = NINP // LANES


def _cbow_body(inp_hbm, table_hbm, out_hbm, idx_v, buf0, buf1, acc, sem0, sem1):
    wid = lax.axis_index("s") * 2 + lax.axis_index("c")
    base = wid * BPW

    pltpu.sync_copy(inp_hbm.at[:, pl.ds(base, BPW)], idx_v)

    zeros = jnp.zeros((LANES,), jnp.float32)

    def zero_body(i, _):
        for j in range(CHUNKS):
            acc[i, pl.ds(j * LANES, LANES)] = zeros
        return 0

    lax.fori_loop(0, BPW, zero_body, 0)

    pltpu.async_copy(table_hbm.at[idx_v.at[0]], buf0, sem0)
    pltpu.async_copy(table_hbm.at[idx_v.at[1]], buf1, sem1)

    def accumulate(buf):
        def row_body(i, _):
            for j in range(CHUNKS):
                sl = pl.ds(j * LANES, LANES)
                plsc.addupdate(acc.at[i, sl], buf[i, sl])
            return 0

        lax.fori_loop(0, BPW, row_body, 0)

    def step(i, _):
        pltpu.make_async_copy(table_hbm.at[idx_v.at[0]], buf0, sem0).wait()
        accumulate(buf0)

        @pl.when(i < (CTX // 2) - 1)
        def _():
            pltpu.async_copy(table_hbm.at[idx_v.at[2 * i + 2]], buf0, sem0)

        pltpu.make_async_copy(table_hbm.at[idx_v.at[1]], buf1, sem1).wait()
        accumulate(buf1)

        @pl.when(i < (CTX // 2) - 1)
        def _():
            pltpu.async_copy(table_hbm.at[idx_v.at[2 * i + 3]], buf1, sem1)

        return 0

    lax.fori_loop(0, CTX // 2, step, 0)

    pltpu.sync_copy(acc, out_hbm.at[pl.ds(base, BPW)])


@jax.jit
def _cbow(inp, table):
    mesh = plsc.VectorSubcoreMesh(core_axis_name="c", subcore_axis_name="s")
    return pl.kernel(
        _cbow_body,
        out_type=jax.ShapeDtypeStruct((BATCH, NINP), jnp.float32),
        mesh=mesh,
        scratch_types=[
            pltpu.VMEM((CTX, BPW), jnp.int32),
            pltpu.VMEM((BPW, NINP), jnp.float32),
            pltpu.VMEM((BPW, NINP), jnp.float32),
            pltpu.VMEM((BPW, NINP), jnp.float32),
            pltpu.SemaphoreType.DMA,
            pltpu.SemaphoreType.DMA,
        ],
        compiler_params=pltpu.CompilerParams(use_tc_tiling_on_sc=False),
    )(inp, table)


def kernel(input, hidden, table):
    del hidden
    return _cbow(input.astype(jnp.int32), table)
